# baseline (speedup 1.0000x reference)
import functools

import jax
import jax.numpy as jnp
from jax import lax
from jax.experimental import pallas as pl
from jax.experimental.pallas import tpu as pltpu
from jax.experimental.pallas import tpu_sc as plsc

N = 10000
NPAD = 10240
E = 160000
ROW_BLK = 512

NC, NS = 2, 16
NW = NC * NS
WRANGE = NPAD // NW
BKT = WRANGE // 2
CAPE = 163840
CHUNK = 2000
FLUSH = 1024
STAGE = 1088
CE = 128
NSLOT = 2
ACCR = BKT + 1

@functools.lru_cache(maxsize=None)
def _sc_mesh():
    return plsc.VectorSubcoreMesh(core_axis_name="c", subcore_axis_name="s",
                                  num_cores=NC, num_subcores=NS)


def _pre_body(nb, h_ref, wt_ref, wb_ref, b_ref, a_ref, *bo_refs):
    hb = h_ref[...]
    a_ref[...] = (jnp.dot(hb, wt_ref[...], preferred_element_type=jnp.float32)
                  + b_ref[...])
    bfull = jnp.dot(hb, wb_ref[...], preferred_element_type=jnp.float32)
    for j in range(nb):
        bo_refs[j][...] = bfull[:, j * 128:(j + 1) * 128]


def _pre_tc(h, wt, wb, b):
    f_in = h.shape[1]
    f = wt.shape[1]
    nb = f // 128
    grid = NPAD // ROW_BLK
    row = lambda i: (i, 0)
    outs = pl.pallas_call(
        functools.partial(_pre_body, nb),
        grid=(grid,),
        in_specs=[
            pl.BlockSpec((ROW_BLK, f_in), row),
            pl.BlockSpec((f_in, f), lambda i: (0, 0)),
            pl.BlockSpec((f_in, f), lambda i: (0, 0)),
            pl.BlockSpec((1, f), lambda i: (0, 0)),
        ],
        out_specs=[pl.BlockSpec((ROW_BLK, f), row)]
        + [pl.BlockSpec((ROW_BLK, 128), row)] * nb,
        out_shape=[jax.ShapeDtypeStruct((NPAD, f), jnp.float32)]
        + [jax.ShapeDtypeStruct((NPAD, 128), jnp.float32)] * nb,
    )(h, wt, wb, b.reshape(1, f))
    return outs[0], outs[1:]


def _scales_body(deg_ref, sc1_ref, sc2_ref):
    deg = deg_ref[...]
    nid = lax.broadcasted_iota(jnp.int32, deg.shape, 0) * 128 + \
        lax.broadcasted_iota(jnp.int32, deg.shape, 1)
    valid = nid < N
    avg_log = jnp.sum(jnp.where(valid, jnp.log(deg + 1.0), 0.0)) / N
    scale = jnp.log(jnp.maximum(deg, 1.0) + 1.0)
    sc1_ref[...] = scale / avg_log
    sc2_ref[...] = avg_log / scale


def _scales_tc(deg):
    deg2 = deg.reshape(NPAD // 128, 128)
    sc1, sc2 = pl.pallas_call(
        _scales_body,
        out_shape=[jax.ShapeDtypeStruct(deg2.shape, jnp.float32)] * 2,
    )(deg2)
    return sc1.reshape(NPAD, 1), sc2.reshape(NPAD, 1)


def _post_body(nf_out, final, h_ref, a_ref, s1_ref, s2_ref, mn_ref, mx_ref,
               deg_ref, sc1_ref, sc2_ref, pw_ref, pb_ref, lw_ref, lb_ref,
               g_ref, be_ref, *rest):
    if final:
        cw_ref, cb_ref, res_ref = rest[0], rest[1], None
        pna_ref, bn_ref, logit_ref = rest[2], rest[3], rest[4]
    else:
        res_ref = rest[0] if len(rest) == 2 else None
        out_ref = rest[-1]
    deg = deg_ref[...]
    degc = jnp.maximum(deg, 1.0)
    a = a_ref[...]
    s1 = s1_ref[...]
    s2 = s2_ref[...]
    s = deg * a + s1
    q = deg * a * a + 2.0 * a * s1 + s2
    mean = s / degc
    var = jnp.maximum(q / degc - mean * mean, 0.0)
    std = jnp.sqrt(var + 1e-5)
    has = deg > 0.0
    mn = jnp.where(has, a + mn_ref[...], 0.0)
    mx = jnp.where(has, a + mx_ref[...], 0.0)
    aggr = jnp.concatenate([mean, mn, mx, std], axis=-1)
    cat = jnp.concatenate(
        [h_ref[...], aggr, aggr * sc1_ref[...], aggr * sc2_ref[...]], axis=-1)
    o = jnp.dot(cat, pw_ref[...], preferred_element_type=jnp.float32) + pb_ref[...]
    o = jnp.dot(o, lw_ref[...], preferred_element_type=jnp.float32) + lb_ref[...]
    mu = jnp.mean(o, axis=-1, keepdims=True)
    xc = o - mu
    v = jnp.mean(xc * xc, axis=-1, keepdims=True)
    ln = xc / jnp.sqrt(v + 1e-5) * g_ref[...] + be_ref[...]
    elu = jnp.where(ln > 0.0, ln, jnp.exp(jnp.minimum(ln, 0.0)) - 1.0)
    if final:
        pna_ref[...] = o
        bn_ref[...] = ln
        logit_ref[...] = (jnp.dot(elu, cw_ref[...],
                                  preferred_element_type=jnp.float32)
                          + cb_ref[...])
    else:
        if res_ref is not None:
            out_ref[...] = elu + res_ref[...]
        else:
            out_ref[...] = elu


def _post_tc(h, a, s1, s2, mn, mx, deg_col, sc1, sc2, conv_p, ln_p,
             residual=None, classifier=None):
    f = a.shape[1]
    f_out = conv_p['post']['W'].shape[1]
    final = classifier is not None
    grid = NPAD // ROW_BLK
    row = lambda i: (i, 0)
    fixed = lambda i: (0, 0)
    in_specs = [
        pl.BlockSpec((ROW_BLK, f), row),
        pl.BlockSpec((ROW_BLK, f), row),
        pl.BlockSpec((ROW_BLK, f), row),
        pl.BlockSpec((ROW_BLK, f), row),
        pl.BlockSpec((ROW_BLK, f), row),
        pl.BlockSpec((ROW_BLK, f), row),
        pl.BlockSpec((ROW_BLK, 1), row),
        pl.BlockSpec((ROW_BLK, 1), row),
        pl.BlockSpec((ROW_BLK, 1), row),
        pl.BlockSpec((13 * f, f_out), fixed),
        pl.BlockSpec((1, f_out), fixed),
        pl.BlockSpec((f_out, f_out), fixed),
        pl.BlockSpec((1, f_out), fixed),
        pl.BlockSpec((1, f_out), fixed),
        pl.BlockSpec((1, f_out), fixed),
    ]
    args = [h, a, s1, s2, mn, mx, deg_col, sc1, sc2,
            conv_p['post']['W'], conv_p['post']['b'].reshape(1, f_out),
            conv_p['lin']['W'], conv_p['lin']['b'].reshape(1, f_out),
            ln_p['gamma'].reshape(1, f_out), ln_p['beta'].reshape(1, f_out)]
    if final:
        ncls = classifier['W'].shape[1]
        in_specs += [pl.BlockSpec((f_out, ncls), fixed),
                     pl.BlockSpec((1, ncls), fixed)]
        args += [classifier['W'], classifier['b'].reshape(1, ncls)]
        out_specs = [pl.BlockSpec((ROW_BLK, f_out), row),
                     pl.BlockSpec((ROW_BLK, f_out), row),
                     pl.BlockSpec((ROW_BLK, ncls), row)]
        out_shape = [jax.ShapeDtypeStruct((NPAD, f_out), jnp.float32),
                     jax.ShapeDtypeStruct((NPAD, f_out), jnp.float32),
                     jax.ShapeDtypeStruct((NPAD, ncls), jnp.float32)]
    else:
        if residual is not None:
            in_specs.append(pl.BlockSpec((ROW_BLK, f_out), row))
            args.append(residual)
        out_specs = pl.BlockSpec((ROW_BLK, f_out), row)
        out_shape = jax.ShapeDtypeStruct((NPAD, f_out), jnp.float32)
    body = functools.partial(_post_body, f_out, final)
    return pl.pallas_call(
        body, grid=(grid,), in_specs=in_specs, out_specs=out_specs,
        out_shape=out_shape,
    )(*args)


def _sc_build_body(src_hbm, dst_hbm, elist, cnt,
                   sbuf, dbuf, stage0, stage1, cvec):
    w = lax.axis_index("s") * NC + lax.axis_index("c")
    base = w * WRANGE
    lo1 = base + BKT
    hi = base + WRANGE
    lane = lax.iota(jnp.int32, 16)

    def do_flush(stage, n, nf, front):
        pred = n >= FLUSH
        @pl.when(pred)
        def _():
            if front:
                pltpu.sync_copy(stage.at[pl.ds(0, FLUSH)],
                                elist.at[w, pl.ds(nf * FLUSH, FLUSH)])
            else:
                pltpu.sync_copy(stage.at[pl.ds(0, FLUSH)],
                                elist.at[w, pl.ds(CAPE - (nf + 1) * FLUSH,
                                                  FLUSH)])
            left = stage[pl.ds(FLUSH, 16)]
            stage[pl.ds(0, 16)] = left
        n = jnp.where(pred, n - FLUSH, n)
        nf = jnp.where(pred, nf + 1, nf)
        return n, nf

    def chunk_body(c, carry):
        off = c * CHUNK
        pltpu.sync_copy(dst_hbm.at[pl.ds(off, CHUNK)], dbuf)
        pltpu.sync_copy(src_hbm.at[pl.ds(off, CHUNK)], sbuf)

        def grp(g, carry):
            n0, nf0, n1, nf1 = carry
            d = dbuf[pl.ds(g * 16, 16)]
            s = sbuf[pl.ds(g * 16, 16)]
            dl = d - base
            pk = (s << 9) | dl
            m0 = (d >= base) & (d < lo1)
            m1 = (d >= lo1) & (d < hi)
            mi0 = jnp.where(m0, 1, 0)
            mi1 = jnp.where(m1, 1, 0)
            for l in range(16):
                pv = jnp.full((16,), pk[l], jnp.int32)
                stage0[pl.ds(n0, 16)] = pv
                n0 = n0 + mi0[l]
                stage1[pl.ds(n1, 16)] = pv
                n1 = n1 + mi1[l]
            n0, nf0 = do_flush(stage0, n0, nf0, True)
            n1, nf1 = do_flush(stage1, n1, nf1, False)
            return n0, nf0, n1, nf1

        return lax.fori_loop(0, CHUNK // 16, grp, carry)

    z = jnp.int32(0)
    n0, nf0, n1, nf1 = lax.fori_loop(
        0, E // CHUNK, chunk_body, (z, z, z, z))

    def pad_tail(stage, n, nf, dummy, front):
        @pl.when(n > 0)
        def _():
            dv = jnp.full((16,), dummy, jnp.int32)
            g0 = n // 16
            rem = n - g0 * 16
            cur = stage[pl.ds(g0 * 16, 16)]
            stage[pl.ds(g0 * 16, 16)] = jnp.where(lane < rem, cur, dv)

            def padg(g, _):
                stage[pl.ds(g * 16, 16)] = dv
                return 0
            lax.fori_loop(g0 + 1, FLUSH // 16, padg, 0)
            if front:
                pltpu.sync_copy(stage.at[pl.ds(0, FLUSH)],
                                elist.at[w, pl.ds(nf * FLUSH, FLUSH)])
            else:
                pltpu.sync_copy(stage.at[pl.ds(0, FLUSH)],
                                elist.at[w, pl.ds(CAPE - (nf + 1) * FLUSH,
                                                  FLUSH)])
        return jnp.where(n > 0, nf + 1, nf)

    nf0 = pad_tail(stage0, n0, nf0, BKT, True)
    nf1 = pad_tail(stage1, n1, nf1, 2 * BKT, False)
    stored0 = nf0 * FLUSH
    stored1 = nf1 * FLUSH

    cvec[...] = jnp.where(lane == 0, stored0,
                          jnp.where(lane == 1, stored1, 0))
    pltpu.sync_copy(cvec, cnt.at[w])


def _sc_build(src, dst):
    k = pl.kernel(
        _sc_build_body,
        out_type=[
            jax.ShapeDtypeStruct((NW, CAPE), jnp.int32),
            jax.ShapeDtypeStruct((NW, 16), jnp.int32),
        ],
        mesh=_sc_mesh(),
        scratch_types=[
            pltpu.VMEM((CHUNK,), jnp.int32),
            pltpu.VMEM((CHUNK,), jnp.int32),
            pltpu.VMEM((STAGE,), jnp.int32),
            pltpu.VMEM((STAGE,), jnp.int32),
            pltpu.VMEM((16,), jnp.int32),
        ],
        compiler_params=pltpu.CompilerParams(use_tc_tiling_on_sc=False),
    )
    return k(src, dst)


def _sc_reduce_body(with_deg, b_hbm, elist, cnt, *rest):
    outs = rest[:5] if with_deg else rest[:4]
    rest = rest[(5 if with_deg else 4):]
    ebuf, ibuf, d2, rbuf = rest[:4]
    accs = rest[4:36]
    rest = rest[36:]
    if with_deg:
        dcnt = rest[0]
        rest = rest[1:]
    cbuf = rest[0]
    sems = rest[1:1 + NSLOT]
    s1r, s2r, mnr, mxr = (accs[0:8], accs[8:16], accs[16:24], accs[24:32])
    w = lax.axis_index("s") * NC + lax.axis_index("c")
    base = w * WRANGE
    pltpu.sync_copy(cnt.at[w], cbuf)
    cv = cbuf[...]
    zf = jnp.zeros((16,), jnp.float32)
    onesf = jnp.ones((16,), jnp.float32)
    big = jnp.full((16,), 1e30, jnp.float32)

    for r in (0, 1):
        stored = pl.multiple_of(cv[r], FLUSH)

        def zb(i, _):
            o = i * 16
            for gg in range(8):
                s1r[gg][pl.ds(o, 16)] = zf
                s2r[gg][pl.ds(o, 16)] = zf
                mnr[gg][pl.ds(o, 16)] = big
                mxr[gg][pl.ds(o, 16)] = -big
            return 0
        lax.fori_loop(0, ACCR, zb, 0)
        if with_deg:
            def zd(i, _):
                dcnt[pl.ds(i * 16, 16)] = zf
                return 0
            lax.fori_loop(0, ACCR, zd, 0)

        nch = stored // CE

        def start_gather(cidx, slot):
            if r == 0:
                off = cidx * CE
            else:
                off = CAPE - stored + cidx * CE
            pltpu.sync_copy(elist.at[w, pl.ds(off, CE)], ebuf)
            so = slot * CE

            def ext(g, _):
                p = ebuf[pl.ds(g * 16, 16)]
                ibuf[pl.ds(so + g * 16, 16)] = p >> 9
                d2[pl.ds(so + g * 16, 16)] = (p & 511) - (r * BKT)
                return 0
            lax.fori_loop(0, CE // 16, ext, 0)
            return pltpu.make_async_copy(
                b_hbm.at[ibuf.at[pl.ds(so, CE)]],
                rbuf.at[pl.ds(slot * CE, CE)], sems[slot])

        for s0 in range(NSLOT):
            @pl.when(s0 < nch)
            def _(s0=s0):
                start_gather(s0, s0).start()

        def do_chunk(cidx, s):
            pltpu.make_async_copy(
                b_hbm.at[ibuf.at[pl.ds(s * CE, CE)]],
                rbuf.at[pl.ds(s * CE, CE)], sems[s]).wait()
            so = s * CE

            def edge_loop(e, _):
                o = d2[pl.ds(so + e, 16)][0] * 16
                for gg in range(8):
                    bv = rbuf[so + e, pl.ds(gg * 16, 16)]
                    s1r[gg][pl.ds(o, 16)] = s1r[gg][pl.ds(o, 16)] + bv
                    s2r[gg][pl.ds(o, 16)] = (s2r[gg][pl.ds(o, 16)]
                                             + bv * bv)
                    mnr[gg][pl.ds(o, 16)] = jnp.minimum(
                        mnr[gg][pl.ds(o, 16)], bv)
                    mxr[gg][pl.ds(o, 16)] = jnp.maximum(
                        mxr[gg][pl.ds(o, 16)], bv)
                if with_deg:
                    dcnt[pl.ds(o, 16)] = dcnt[pl.ds(o, 16)] + onesf
                return 0
            lax.fori_loop(0, CE, edge_loop, 0)
            @pl.when(cidx + NSLOT < nch)
            def _():
                start_gather(cidx + NSLOT, s).start()

        def chunkn(ci, _):
            for s in range(NSLOT):
                cidx = ci * NSLOT + s

                @pl.when(cidx < nch)
                def _(cidx=cidx, s=s):
                    do_chunk(cidx, s)
            return 0

        lax.fori_loop(0, (nch + NSLOT - 1) // NSLOT, chunkn, 0)
        out_off = (base + r * BKT) * 16
        for st, refs in enumerate((s1r, s2r, mnr, mxr)):
            for gg in range(8):
                pltpu.sync_copy(refs[gg].at[pl.ds(0, BKT * 16)],
                                outs[st].at[gg, pl.ds(out_off, BKT * 16)])
        if with_deg:
            pltpu.sync_copy(dcnt.at[pl.ds(0, BKT * 16)],
                            outs[4].at[pl.ds(out_off, BKT * 16)])


def _sc_reduce(b_mat, elist, cnt, with_deg=False):
    out_type = [jax.ShapeDtypeStruct((8, NPAD * 16), jnp.float32)] * 4
    if with_deg:
        out_type.append(jax.ShapeDtypeStruct((NPAD * 16,), jnp.float32))
    scratch = [
        pltpu.VMEM((CE,), jnp.int32),
        pltpu.VMEM((NSLOT * CE,), jnp.int32),
        pltpu.VMEM((NSLOT * CE + 16,), jnp.int32),
        pltpu.VMEM((NSLOT * CE, 128), jnp.float32),
    ]
    scratch += [pltpu.VMEM((ACCR * 16,), jnp.float32)] * 32
    if with_deg:
        scratch.append(pltpu.VMEM((ACCR * 16,), jnp.float32))
    scratch += [pltpu.VMEM((16,), jnp.int32)]
    scratch += [pltpu.SemaphoreType.DMA] * NSLOT
    k = pl.kernel(
        functools.partial(_sc_reduce_body, with_deg),
        out_type=out_type,
        mesh=_sc_mesh(),
        scratch_types=scratch,
        compiler_params=pltpu.CompilerParams(use_tc_tiling_on_sc=False),
    )
    outs = k(b_mat, elist, cnt)
    res = [o.reshape(8, NPAD, 16).transpose(1, 0, 2).reshape(NPAD, 128)
           for o in outs[:4]]
    if with_deg:
        res.append(outs[4].reshape(NPAD, 16)[:, 0])
    return res


def _segment_stats(b_chunks, elist, cnt, with_deg=False):
    s1s, s2s, mns, mxs = [], [], [], []
    deg = None
    for j, b in enumerate(b_chunks):
        wd = with_deg and j == 0
        outs = _sc_reduce(b, elist, cnt, with_deg=wd)
        s1s.append(outs[0])
        s2s.append(outs[1])
        mns.append(outs[2])
        mxs.append(outs[3])
        if wd:
            deg = outs[4]
    cat = (lambda lst: lst[0] if len(lst) == 1
           else jnp.concatenate(lst, axis=1))
    return cat(s1s), cat(s2s), cat(mns), cat(mxs), deg


def kernel(x, edge_index, params):
    src, dst = edge_index[0], edge_index[1]
    xp = jnp.pad(x, ((0, NPAD - N), (0, 0)))
    elist, cnt = _sc_build(src, dst)
    state = {}

    def layer(h, cp, lnp, residual=None, classifier=None, first=False):
        f_in = h.shape[1]
        a, b_chunks = _pre_tc(h, cp['pre']['W'][:f_in], cp['pre']['W'][f_in:],
                              cp['pre']['b'])
        s1, s2, mn, mx, deg = _segment_stats(b_chunks, elist, cnt,
                                             with_deg=first)
        if first:
            state['deg_col'] = deg.reshape(NPAD, 1)
            state['sc1'], state['sc2'] = _scales_tc(deg)
        return _post_tc(h, a, s1, s2, mn, mx, state['deg_col'],
                        state['sc1'], state['sc2'], cp, lnp,
                        residual=residual, classifier=classifier)

    p = params
    h1 = layer(xp, p['conv1'], p['bn1'], first=True)
    h2 = layer(h1, p['conv2'], p['bn2'])
    h4_in = layer(h2, p['conv3'], p['bn3'], residual=h1)
    out_pna, out_bn, logits = layer(h4_in, p['conv4'], p['bn4'],
                                    classifier=p['classifier'])
    return (logits[:N], out_pna[:N], out_bn[:N])


# speedup vs baseline: 1.0210x; 1.0210x over previous
import functools

import jax
import jax.numpy as jnp
from jax import lax
from jax.experimental import pallas as pl
from jax.experimental.pallas import tpu as pltpu
from jax.experimental.pallas import tpu_sc as plsc

N = 10000
NPAD = 10240
E = 160000
ROW_BLK = 512

NC, NS = 2, 16
NW = NC * NS
WRANGE = NPAD // NW
BKT = WRANGE // 2
CAPE = 163840
CHUNK = 2000
FLUSH = 1024
STAGE = 1088
CE = 128
NSLOT = 2
ACCR = BKT + 1

@functools.lru_cache(maxsize=None)
def _sc_mesh():
    return plsc.VectorSubcoreMesh(core_axis_name="c", subcore_axis_name="s",
                                  num_cores=NC, num_subcores=NS)


def _pre_body(nb, h_ref, wt_ref, wb_ref, b_ref, a_ref, *bo_refs):
    hb = h_ref[...]
    a_ref[...] = (jnp.dot(hb, wt_ref[...], preferred_element_type=jnp.float32)
                  + b_ref[...])
    bfull = jnp.dot(hb, wb_ref[...], preferred_element_type=jnp.float32)
    for j in range(nb):
        bo_refs[j][...] = bfull[:, j * 128:(j + 1) * 128]


def _pre_tc(h, wt, wb, b):
    f_in = h.shape[1]
    f = wt.shape[1]
    nb = f // 128
    grid = NPAD // ROW_BLK
    row = lambda i: (i, 0)
    outs = pl.pallas_call(
        functools.partial(_pre_body, nb),
        grid=(grid,),
        in_specs=[
            pl.BlockSpec((ROW_BLK, f_in), row),
            pl.BlockSpec((f_in, f), lambda i: (0, 0)),
            pl.BlockSpec((f_in, f), lambda i: (0, 0)),
            pl.BlockSpec((1, f), lambda i: (0, 0)),
        ],
        out_specs=[pl.BlockSpec((ROW_BLK, f), row)]
        + [pl.BlockSpec((ROW_BLK, 128), row)] * nb,
        out_shape=[jax.ShapeDtypeStruct((NPAD, f), jnp.float32)]
        + [jax.ShapeDtypeStruct((NPAD, 128), jnp.float32)] * nb,
    )(h, wt, wb, b.reshape(1, f))
    return outs[0], outs[1:]


def _scales_body(deg_ref, sc1_ref, sc2_ref):
    deg = deg_ref[...]
    nid = lax.broadcasted_iota(jnp.int32, deg.shape, 0) * 128 + \
        lax.broadcasted_iota(jnp.int32, deg.shape, 1)
    valid = nid < N
    avg_log = jnp.sum(jnp.where(valid, jnp.log(deg + 1.0), 0.0)) / N
    scale = jnp.log(jnp.maximum(deg, 1.0) + 1.0)
    sc1_ref[...] = scale / avg_log
    sc2_ref[...] = avg_log / scale


def _scales_tc(deg):
    deg2 = deg.reshape(NPAD // 128, 128)
    sc1, sc2 = pl.pallas_call(
        _scales_body,
        out_shape=[jax.ShapeDtypeStruct(deg2.shape, jnp.float32)] * 2,
    )(deg2)
    return sc1.reshape(NPAD, 1), sc2.reshape(NPAD, 1)


def _post_body(nf_out, final, h_ref, a_ref, s1_ref, s2_ref, mn_ref, mx_ref,
               deg_ref, sc1_ref, sc2_ref, pw_ref, pb_ref, lw_ref, lb_ref,
               g_ref, be_ref, *rest):
    if final:
        cw_ref, cb_ref, res_ref = rest[0], rest[1], None
        pna_ref, bn_ref, logit_ref = rest[2], rest[3], rest[4]
    else:
        res_ref = rest[0] if len(rest) == 2 else None
        out_ref = rest[-1]
    deg = deg_ref[...]
    degc = jnp.maximum(deg, 1.0)
    a = a_ref[...]
    s1 = s1_ref[...]
    s2 = s2_ref[...]
    s = deg * a + s1
    q = deg * a * a + 2.0 * a * s1 + s2
    mean = s / degc
    var = jnp.maximum(q / degc - mean * mean, 0.0)
    std = jnp.sqrt(var + 1e-5)
    has = deg > 0.0
    mn = jnp.where(has, a + mn_ref[...], 0.0)
    mx = jnp.where(has, a + mx_ref[...], 0.0)
    aggr = jnp.concatenate([mean, mn, mx, std], axis=-1)
    cat = jnp.concatenate(
        [h_ref[...], aggr, aggr * sc1_ref[...], aggr * sc2_ref[...]], axis=-1)
    o = jnp.dot(cat, pw_ref[...], preferred_element_type=jnp.float32) + pb_ref[...]
    o = jnp.dot(o, lw_ref[...], preferred_element_type=jnp.float32) + lb_ref[...]
    mu = jnp.mean(o, axis=-1, keepdims=True)
    xc = o - mu
    v = jnp.mean(xc * xc, axis=-1, keepdims=True)
    ln = xc / jnp.sqrt(v + 1e-5) * g_ref[...] + be_ref[...]
    elu = jnp.where(ln > 0.0, ln, jnp.exp(jnp.minimum(ln, 0.0)) - 1.0)
    if final:
        pna_ref[...] = o
        bn_ref[...] = ln
        logit_ref[...] = (jnp.dot(elu, cw_ref[...],
                                  preferred_element_type=jnp.float32)
                          + cb_ref[...])
    else:
        if res_ref is not None:
            out_ref[...] = elu + res_ref[...]
        else:
            out_ref[...] = elu


def _post_tc(h, a, s1, s2, mn, mx, deg_col, sc1, sc2, conv_p, ln_p,
             residual=None, classifier=None):
    f = a.shape[1]
    f_out = conv_p['post']['W'].shape[1]
    final = classifier is not None
    grid = NPAD // ROW_BLK
    row = lambda i: (i, 0)
    fixed = lambda i: (0, 0)
    in_specs = [
        pl.BlockSpec((ROW_BLK, f), row),
        pl.BlockSpec((ROW_BLK, f), row),
        pl.BlockSpec((ROW_BLK, f), row),
        pl.BlockSpec((ROW_BLK, f), row),
        pl.BlockSpec((ROW_BLK, f), row),
        pl.BlockSpec((ROW_BLK, f), row),
        pl.BlockSpec((ROW_BLK, 1), row),
        pl.BlockSpec((ROW_BLK, 1), row),
        pl.BlockSpec((ROW_BLK, 1), row),
        pl.BlockSpec((13 * f, f_out), fixed),
        pl.BlockSpec((1, f_out), fixed),
        pl.BlockSpec((f_out, f_out), fixed),
        pl.BlockSpec((1, f_out), fixed),
        pl.BlockSpec((1, f_out), fixed),
        pl.BlockSpec((1, f_out), fixed),
    ]
    args = [h, a, s1, s2, mn, mx, deg_col, sc1, sc2,
            conv_p['post']['W'], conv_p['post']['b'].reshape(1, f_out),
            conv_p['lin']['W'], conv_p['lin']['b'].reshape(1, f_out),
            ln_p['gamma'].reshape(1, f_out), ln_p['beta'].reshape(1, f_out)]
    if final:
        ncls = classifier['W'].shape[1]
        in_specs += [pl.BlockSpec((f_out, ncls), fixed),
                     pl.BlockSpec((1, ncls), fixed)]
        args += [classifier['W'], classifier['b'].reshape(1, ncls)]
        out_specs = [pl.BlockSpec((ROW_BLK, f_out), row),
                     pl.BlockSpec((ROW_BLK, f_out), row),
                     pl.BlockSpec((ROW_BLK, ncls), row)]
        out_shape = [jax.ShapeDtypeStruct((NPAD, f_out), jnp.float32),
                     jax.ShapeDtypeStruct((NPAD, f_out), jnp.float32),
                     jax.ShapeDtypeStruct((NPAD, ncls), jnp.float32)]
    else:
        if residual is not None:
            in_specs.append(pl.BlockSpec((ROW_BLK, f_out), row))
            args.append(residual)
        out_specs = pl.BlockSpec((ROW_BLK, f_out), row)
        out_shape = jax.ShapeDtypeStruct((NPAD, f_out), jnp.float32)
    body = functools.partial(_post_body, f_out, final)
    return pl.pallas_call(
        body, grid=(grid,), in_specs=in_specs, out_specs=out_specs,
        out_shape=out_shape,
    )(*args)


def _sc_build_body(src_hbm, dst_hbm, elist, cnt,
                   sbuf, dbuf, stage0, stage1, cvec):
    w = lax.axis_index("s") * NC + lax.axis_index("c")
    base = w * WRANGE
    lo1 = base + BKT
    hi = base + WRANGE
    lane = lax.iota(jnp.int32, 16)

    def do_flush(stage, n, nf, front):
        pred = n >= FLUSH
        @pl.when(pred)
        def _():
            if front:
                pltpu.sync_copy(stage.at[pl.ds(0, FLUSH)],
                                elist.at[w, pl.ds(nf * FLUSH, FLUSH)])
            else:
                pltpu.sync_copy(stage.at[pl.ds(0, FLUSH)],
                                elist.at[w, pl.ds(CAPE - (nf + 1) * FLUSH,
                                                  FLUSH)])
            left = stage[pl.ds(FLUSH, 16)]
            stage[pl.ds(0, 16)] = left
        n = jnp.where(pred, n - FLUSH, n)
        nf = jnp.where(pred, nf + 1, nf)
        return n, nf

    def chunk_body(c, carry):
        off = c * CHUNK
        pltpu.sync_copy(dst_hbm.at[pl.ds(off, CHUNK)], dbuf)
        pltpu.sync_copy(src_hbm.at[pl.ds(off, CHUNK)], sbuf)

        def grp(g, carry):
            n0, nf0, n1, nf1 = carry
            d = dbuf[pl.ds(g * 16, 16)]
            s = sbuf[pl.ds(g * 16, 16)]
            dl = d - base
            pk = (s << 9) | dl
            m0 = (d >= base) & (d < lo1)
            m1 = (d >= lo1) & (d < hi)
            mi0 = jnp.where(m0, 1, 0)
            mi1 = jnp.where(m1, 1, 0)
            for l in range(16):
                pv = jnp.full((16,), pk[l], jnp.int32)
                stage0[pl.ds(n0, 16)] = pv
                n0 = n0 + mi0[l]
                stage1[pl.ds(n1, 16)] = pv
                n1 = n1 + mi1[l]
            n0, nf0 = do_flush(stage0, n0, nf0, True)
            n1, nf1 = do_flush(stage1, n1, nf1, False)
            return n0, nf0, n1, nf1

        return lax.fori_loop(0, CHUNK // 16, grp, carry)

    z = jnp.int32(0)
    n0, nf0, n1, nf1 = lax.fori_loop(
        0, E // CHUNK, chunk_body, (z, z, z, z))

    def pad_tail(stage, n, nf, dummy, front):
        @pl.when(n > 0)
        def _():
            dv = jnp.full((16,), dummy, jnp.int32)
            g0 = n // 16
            rem = n - g0 * 16
            cur = stage[pl.ds(g0 * 16, 16)]
            stage[pl.ds(g0 * 16, 16)] = jnp.where(lane < rem, cur, dv)

            def padg(g, _):
                stage[pl.ds(g * 16, 16)] = dv
                return 0
            lax.fori_loop(g0 + 1, FLUSH // 16, padg, 0)
            if front:
                pltpu.sync_copy(stage.at[pl.ds(0, FLUSH)],
                                elist.at[w, pl.ds(nf * FLUSH, FLUSH)])
            else:
                pltpu.sync_copy(stage.at[pl.ds(0, FLUSH)],
                                elist.at[w, pl.ds(CAPE - (nf + 1) * FLUSH,
                                                  FLUSH)])
        return jnp.where(n > 0, nf + 1, nf)

    nf0 = pad_tail(stage0, n0, nf0, BKT, True)
    nf1 = pad_tail(stage1, n1, nf1, 2 * BKT, False)
    stored0 = nf0 * FLUSH
    stored1 = nf1 * FLUSH

    cvec[...] = jnp.where(lane == 0, stored0,
                          jnp.where(lane == 1, stored1, 0))
    pltpu.sync_copy(cvec, cnt.at[w])


def _sc_build(src, dst):
    k = pl.kernel(
        _sc_build_body,
        out_type=[
            jax.ShapeDtypeStruct((NW, CAPE), jnp.int32),
            jax.ShapeDtypeStruct((NW, 16), jnp.int32),
        ],
        mesh=_sc_mesh(),
        scratch_types=[
            pltpu.VMEM((CHUNK,), jnp.int32),
            pltpu.VMEM((CHUNK,), jnp.int32),
            pltpu.VMEM((STAGE,), jnp.int32),
            pltpu.VMEM((STAGE,), jnp.int32),
            pltpu.VMEM((16,), jnp.int32),
        ],
        compiler_params=pltpu.CompilerParams(use_tc_tiling_on_sc=False),
    )
    return k(src, dst)


def _sc_reduce_body(with_deg, b_hbm, elist, cnt, *rest):
    outs = rest[:5] if with_deg else rest[:4]
    rest = rest[(5 if with_deg else 4):]
    ebuf, ibuf, d2, rbuf = rest[:4]
    accs = rest[4:36]
    rest = rest[36:]
    if with_deg:
        dcnt = rest[0]
        rest = rest[1:]
    cbuf = rest[0]
    sems = rest[1:1 + NSLOT]
    s1r, s2r, mnr, mxr = (accs[0:8], accs[8:16], accs[16:24], accs[24:32])
    w = lax.axis_index("s") * NC + lax.axis_index("c")
    base = w * WRANGE
    pltpu.sync_copy(cnt.at[w], cbuf)
    cv = cbuf[...]
    zf = jnp.zeros((16,), jnp.float32)
    onesf = jnp.ones((16,), jnp.float32)
    big = jnp.full((16,), 1e30, jnp.float32)

    for r in (0, 1):
        stored = pl.multiple_of(cv[r], FLUSH)

        def zb(i, _):
            o = i * 16
            for gg in range(8):
                s1r[gg][pl.ds(o, 16)] = zf
                s2r[gg][pl.ds(o, 16)] = zf
                mnr[gg][pl.ds(o, 16)] = big
                mxr[gg][pl.ds(o, 16)] = -big
            return 0
        lax.fori_loop(0, ACCR, zb, 0)
        if with_deg:
            def zd(i, _):
                dcnt[pl.ds(i * 16, 16)] = zf
                return 0
            lax.fori_loop(0, ACCR, zd, 0)

        nch = stored // CE

        def start_gather(cidx, slot):
            if r == 0:
                off = cidx * CE
            else:
                off = CAPE - stored + cidx * CE
            pltpu.sync_copy(elist.at[w, pl.ds(off, CE)], ebuf)
            so = slot * CE

            def ext(g, _):
                p = ebuf[pl.ds(g * 16, 16)]
                ibuf[pl.ds(so + g * 16, 16)] = p >> 9
                d2[pl.ds(so + g * 16, 16)] = (p & 511) - (r * BKT)
                return 0
            lax.fori_loop(0, CE // 16, ext, 0)
            return pltpu.make_async_copy(
                b_hbm.at[ibuf.at[pl.ds(so, CE)]],
                rbuf.at[pl.ds(slot * CE, CE)], sems[slot])

        for s0 in range(NSLOT):
            @pl.when(s0 < nch)
            def _(s0=s0):
                start_gather(s0, s0).start()

        def do_chunk(cidx, s):
            pltpu.make_async_copy(
                b_hbm.at[ibuf.at[pl.ds(s * CE, CE)]],
                rbuf.at[pl.ds(s * CE, CE)], sems[s]).wait()
            so = s * CE

            def grp16(g, _):
                dv = d2[pl.ds(so + g * 16, 16)]
                for l in range(16):
                    o = dv[l] * 16
                    for gg in range(8):
                        bv = rbuf[so + g * 16 + l, pl.ds(gg * 16, 16)]
                        s1r[gg][pl.ds(o, 16)] = s1r[gg][pl.ds(o, 16)] + bv
                        s2r[gg][pl.ds(o, 16)] = (s2r[gg][pl.ds(o, 16)]
                                                 + bv * bv)
                        mnr[gg][pl.ds(o, 16)] = jnp.minimum(
                            mnr[gg][pl.ds(o, 16)], bv)
                        mxr[gg][pl.ds(o, 16)] = jnp.maximum(
                            mxr[gg][pl.ds(o, 16)], bv)
                    if with_deg:
                        dcnt[pl.ds(o, 16)] = dcnt[pl.ds(o, 16)] + onesf
                return 0
            lax.fori_loop(0, CE // 16, grp16, 0)
            @pl.when(cidx + NSLOT < nch)
            def _():
                start_gather(cidx + NSLOT, s).start()

        def chunkn(ci, _):
            for s in range(NSLOT):
                cidx = ci * NSLOT + s

                @pl.when(cidx < nch)
                def _(cidx=cidx, s=s):
                    do_chunk(cidx, s)
            return 0

        lax.fori_loop(0, (nch + NSLOT - 1) // NSLOT, chunkn, 0)
        out_off = (base + r * BKT) * 16
        for st, refs in enumerate((s1r, s2r, mnr, mxr)):
            for gg in range(8):
                pltpu.sync_copy(refs[gg].at[pl.ds(0, BKT * 16)],
                                outs[st].at[gg, pl.ds(out_off, BKT * 16)])
        if with_deg:
            pltpu.sync_copy(dcnt.at[pl.ds(0, BKT * 16)],
                            outs[4].at[pl.ds(out_off, BKT * 16)])


def _sc_reduce(b_mat, elist, cnt, with_deg=False):
    out_type = [jax.ShapeDtypeStruct((8, NPAD * 16), jnp.float32)] * 4
    if with_deg:
        out_type.append(jax.ShapeDtypeStruct((NPAD * 16,), jnp.float32))
    scratch = [
        pltpu.VMEM((CE,), jnp.int32),
        pltpu.VMEM((NSLOT * CE,), jnp.int32),
        pltpu.VMEM((NSLOT * CE + 16,), jnp.int32),
        pltpu.VMEM((NSLOT * CE, 128), jnp.float32),
    ]
    scratch += [pltpu.VMEM((ACCR * 16,), jnp.float32)] * 32
    if with_deg:
        scratch.append(pltpu.VMEM((ACCR * 16,), jnp.float32))
    scratch += [pltpu.VMEM((16,), jnp.int32)]
    scratch += [pltpu.SemaphoreType.DMA] * NSLOT
    k = pl.kernel(
        functools.partial(_sc_reduce_body, with_deg),
        out_type=out_type,
        mesh=_sc_mesh(),
        scratch_types=scratch,
        compiler_params=pltpu.CompilerParams(use_tc_tiling_on_sc=False),
    )
    outs = k(b_mat, elist, cnt)
    res = [o.reshape(8, NPAD, 16).transpose(1, 0, 2).reshape(NPAD, 128)
           for o in outs[:4]]
    if with_deg:
        res.append(outs[4].reshape(NPAD, 16)[:, 0])
    return res


def _segment_stats(b_chunks, elist, cnt, with_deg=False):
    s1s, s2s, mns, mxs = [], [], [], []
    deg = None
    for j, b in enumerate(b_chunks):
        wd = with_deg and j == 0
        outs = _sc_reduce(b, elist, cnt, with_deg=wd)
        s1s.append(outs[0])
        s2s.append(outs[1])
        mns.append(outs[2])
        mxs.append(outs[3])
        if wd:
            deg = outs[4]
    cat = (lambda lst: lst[0] if len(lst) == 1
           else jnp.concatenate(lst, axis=1))
    return cat(s1s), cat(s2s), cat(mns), cat(mxs), deg


def kernel(x, edge_index, params):
    src, dst = edge_index[0], edge_index[1]
    xp = jnp.pad(x, ((0, NPAD - N), (0, 0)))
    elist, cnt = _sc_build(src, dst)
    state = {}

    def layer(h, cp, lnp, residual=None, classifier=None, first=False):
        f_in = h.shape[1]
        a, b_chunks = _pre_tc(h, cp['pre']['W'][:f_in], cp['pre']['W'][f_in:],
                              cp['pre']['b'])
        s1, s2, mn, mx, deg = _segment_stats(b_chunks, elist, cnt,
                                             with_deg=first)
        if first:
            state['deg_col'] = deg.reshape(NPAD, 1)
            state['sc1'], state['sc2'] = _scales_tc(deg)
        return _post_tc(h, a, s1, s2, mn, mx, state['deg_col'],
                        state['sc1'], state['sc2'], cp, lnp,
                        residual=residual, classifier=classifier)

    p = params
    h1 = layer(xp, p['conv1'], p['bn1'], first=True)
    h2 = layer(h1, p['conv2'], p['bn2'])
    h4_in = layer(h2, p['conv3'], p['bn3'], residual=h1)
    out_pna, out_bn, logits = layer(h4_in, p['conv4'], p['bn4'],
                                    classifier=p['classifier'])
    return (logits[:N], out_pna[:N], out_bn[:N])
